# baseline (speedup 1.0000x reference)
import functools

import jax
import jax.numpy as jnp
from jax import lax
from jax.experimental import pallas as pl
from jax.experimental.pallas import tpu as pltpu
from jax.experimental.pallas import tpu_sc as plsc

MAX_DEPTH = 20
EMB = 128
LANES = 16
W = 112
NUM_CORES = 2
NUM_SUBCORES = 16
NW = NUM_CORES * NUM_SUBCORES


def _make_sc_kernel(npad, chunks_per_worker):
    mesh = plsc.VectorSubcoreMesh(core_axis_name="c", subcore_axis_name="s")

    @functools.partial(
        pl.kernel,
        out_type=jax.ShapeDtypeStruct((npad, EMB), jnp.float32),
        mesh=mesh,
        scratch_types=[
            pltpu.VMEM((W,), jnp.int32),
            pltpu.VMEM((W,), jnp.int32),
            pltpu.VMEM((W,), jnp.int32),
            pltpu.VMEM((W, EMB), jnp.float32),
            pltpu.VMEM((W, EMB), jnp.float32),
            pltpu.VMEM((W, EMB), jnp.float32),
            pltpu.SemaphoreType.DMA,
            pltpu.SemaphoreType.DMA,
            pltpu.SemaphoreType.DMA,
        ],
    )
    def sc_kernel(x0_hbm, x1_hbm, d_hbm, tt_hbm, at_hbm, dt_hbm, out_hbm,
                  i0, i1, i2, acc, brows, crows, s0, s1, s2):
        wid = lax.axis_index("s") * NUM_CORES + lax.axis_index("c")

        @pl.loop(0, chunks_per_worker)
        def _chunk(j):
            base = (wid * chunks_per_worker + j) * W
            pltpu.sync_copy(x0_hbm.at[pl.ds(base, W)], i0)
            pltpu.sync_copy(x1_hbm.at[pl.ds(base, W)], i1)
            pltpu.sync_copy(d_hbm.at[pl.ds(base, W)], i2)

            @pl.loop(0, W, step=LANES)
            def _clip(t):
                sl = pl.ds(t, LANES)
                i2[sl] = jnp.minimum(i2[sl], MAX_DEPTH)

            ca = pltpu.async_copy(tt_hbm.at[i0], acc, s0)
            cb = pltpu.async_copy(at_hbm.at[i1], brows, s1)
            cc = pltpu.async_copy(dt_hbm.at[i2], crows, s2)
            ca.wait()
            cb.wait()
            cc.wait()

            @pl.loop(0, W)
            def _row(r):
                for t in range(0, EMB, LANES):
                    sl = pl.ds(t, LANES)
                    plsc.addupdate(acc.at[r, sl], brows[r, sl] + crows[r, sl])

            pltpu.sync_copy(acc, out_hbm.at[pl.ds(base, W)])

    return sc_kernel


def kernel(x, node_depth, type_table, attribute_table, depth_table):
    n = x.shape[0]
    rows_per_round = NW * W
    chunks_per_worker = -(-n // rows_per_round)
    npad = rows_per_round * chunks_per_worker

    x0 = jnp.pad(x[:, 0].astype(jnp.int32), (0, npad - n))
    x1 = jnp.pad(x[:, 1].astype(jnp.int32), (0, npad - n))
    d = jnp.pad(node_depth.reshape(-1).astype(jnp.int32), (0, npad - n))

    sc_kernel = _make_sc_kernel(npad, chunks_per_worker)
    out = sc_kernel(x0, x1, d, type_table, attribute_table, depth_table)
    return out[:n]


# speedup vs baseline: 6.0261x; 6.0261x over previous
import functools

import jax
import jax.numpy as jnp
from jax import lax
from jax.experimental import pallas as pl
from jax.experimental.pallas import tpu as pltpu
from jax.experimental.pallas import tpu_sc as plsc

MAX_DEPTH = 20
NUM_DEPTH = MAX_DEPTH + 1
EMB = 128
LANES = 16
W = 112
NUM_CORES = 2
NUM_SUBCORES = 16
NW = NUM_CORES * NUM_SUBCORES


def _build_combined(type_table, depth_table):
    nt, nd = type_table.shape[0], depth_table.shape[0]

    def body(tt_ref, dt_ref, out_ref):
        out_ref[...] = tt_ref[...][:, None, :] + dt_ref[...][None, :, :]

    return pl.pallas_call(
        body,
        out_shape=jax.ShapeDtypeStruct((nt, nd, EMB), jnp.float32),
    )(type_table, depth_table)


def _make_sc_kernel(npad, chunks_per_worker):
    mesh = plsc.VectorSubcoreMesh(core_axis_name="c", subcore_axis_name="s")
    nchunks = npad // W

    @functools.partial(
        pl.kernel,
        out_type=jax.ShapeDtypeStruct((npad, EMB), jnp.float32),
        mesh=mesh,
        scratch_types=[
            pltpu.VMEM((3, W), jnp.int32),
            pltpu.VMEM((3, W), jnp.int32),
            pltpu.VMEM((W,), jnp.int32),
            pltpu.VMEM((W,), jnp.int32),
            pltpu.VMEM((W,), jnp.int32),
            pltpu.VMEM((W,), jnp.int32),
            pltpu.VMEM((W, EMB), jnp.float32),
            pltpu.VMEM((W, EMB), jnp.float32),
            pltpu.VMEM((W, EMB), jnp.float32),
            pltpu.VMEM((W, EMB), jnp.float32),
            pltpu.SemaphoreType.DMA,
            pltpu.SemaphoreType.DMA,
            pltpu.SemaphoreType.DMA,
            pltpu.SemaphoreType.DMA,
            pltpu.SemaphoreType.DMA,
            pltpu.SemaphoreType.DMA,
        ],
    )
    def sc_kernel(idx_hbm, comb_hbm, attr_hbm, out_hbm,
                  raw0, raw1, ic0, ic1, ia0, ia1, cr0, cr1, ar0, ar1,
                  cg0, cg1, ag0, ag1, ws0, ws1):
        raws, ics, ias = (raw0, raw1), (ic0, ic1), (ia0, ia1)
        crs, ars = (cr0, cr1), (ar0, ar1)
        cgs, ags, wss = (cg0, cg1), (ag0, ag1), (ws0, ws1)

        wid = lax.axis_index("s") * NUM_CORES + lax.axis_index("c")
        first = wid * chunks_per_worker

        def prepare(chunk_id, s, wait_write):
            pltpu.sync_copy(idx_hbm.at[chunk_id], raws[s])

            @pl.loop(0, W, step=LANES)
            def _fuse(t):
                sl = pl.ds(t, LANES)
                d = jnp.minimum(raws[s][1, sl], MAX_DEPTH)
                ics[s][sl] = raws[s][0, sl] * NUM_DEPTH + d
                ias[s][sl] = raws[s][2, sl]

            if wait_write:
                pltpu.make_async_copy(
                    crs[s], out_hbm.at[pl.ds(0, W)], wss[s]).wait()
            pltpu.async_copy(comb_hbm.at[ics[s]], crs[s], cgs[s])
            pltpu.async_copy(attr_hbm.at[ias[s]], ars[s], ags[s])

        def finish(chunk_id, s):
            pltpu.make_async_copy(comb_hbm.at[ics[s]], crs[s], cgs[s]).wait()
            pltpu.make_async_copy(attr_hbm.at[ias[s]], ars[s], ags[s]).wait()

            @pl.loop(0, W, step=2)
            def _rows(r):
                for rr in range(2):
                    for t in range(0, EMB, LANES):
                        sl = pl.ds(t, LANES)
                        plsc.addupdate(crs[s].at[r + rr, sl],
                                       ars[s][r + rr, sl])

            pltpu.async_copy(crs[s], out_hbm.at[pl.ds(chunk_id * W, W)],
                             wss[s])

        prepare(first, 0, False)
        prepare(first + 1, 1, False)

        @pl.loop(0, chunks_per_worker - 2, step=2)
        def _main(j):
            for b in range(2):
                finish(first + j + b, b)
                prepare(first + j + b + 2, b, True)

        finish(first + chunks_per_worker - 2, 0)
        finish(first + chunks_per_worker - 1, 1)
        pltpu.make_async_copy(crs[0], out_hbm.at[pl.ds(0, W)], wss[0]).wait()
        pltpu.make_async_copy(crs[1], out_hbm.at[pl.ds(0, W)], wss[1]).wait()

    return sc_kernel


def kernel(x, node_depth, type_table, attribute_table, depth_table):
    n = x.shape[0]
    rows_per_round = NW * W
    chunks_per_worker = -(-n // rows_per_round)
    chunks_per_worker += chunks_per_worker % 2
    npad = rows_per_round * chunks_per_worker
    nchunks = npad // W

    x0 = jnp.pad(x[:, 0].astype(jnp.int32), (0, npad - n))
    x1 = jnp.pad(x[:, 1].astype(jnp.int32), (0, npad - n))
    d = jnp.pad(node_depth.reshape(-1).astype(jnp.int32), (0, npad - n))
    idx = (jnp.stack([x0, d, x1])
           .reshape(3, nchunks, W)
           .transpose(1, 0, 2))

    comb = _build_combined(type_table, depth_table).reshape(-1, EMB)

    sc_kernel = _make_sc_kernel(npad, chunks_per_worker)
    out = sc_kernel(idx, comb, attribute_table)
    return out[:n]
